# baseline (speedup 1.0000x reference)
import jax
import jax.numpy as jnp
from jax.experimental import pallas as pl
from jax.experimental.pallas import tpu as pltpu

_BLK = 1024


def _vq_kernel(x_ref, w_ref, cbt_ref, out_ref, wt_ref, cb_ref, c2_ref):
    @pl.when(pl.program_id(0) == 0)
    def _():
        wt_ref[...] = w_ref[...].T
        cb = cbt_ref[...].T
        cb_ref[...] = cb
        c2_ref[...] = jnp.sum(cb * cb, axis=1, keepdims=True)

    H = x_ref.shape[0] // 4
    for p in range(4):
        h = jnp.dot(x_ref[p * H:(p + 1) * H, :], wt_ref[...],
                    preferred_element_type=jnp.float32)
        mean = jnp.mean(h, axis=-1, keepdims=True)
        hc = h - mean
        var = jnp.mean(hc * hc, axis=-1, keepdims=True)
        hn = hc * jax.lax.rsqrt(var + 1e-5)
        scores_t = jnp.dot(cb_ref[...], hn.T,
                           preferred_element_type=jnp.float32)
        val = c2_ref[...] - 2.0 * scores_t
        idx = jnp.argmin(val, axis=0)
        out_ref[p * H:(p + 1) * H] = idx.astype(jnp.int32)


@jax.jit
def kernel(x, W, codebook):
    B, L, DIM = x.shape
    K, CD = codebook.shape
    N = B * L
    xf = x.reshape(N, DIM)
    cbt = codebook.T
    grid = (N // _BLK,)
    out = pl.pallas_call(
        _vq_kernel,
        grid=grid,
        in_specs=[
            pl.BlockSpec((_BLK, DIM), lambda i: (i, 0)),
            pl.BlockSpec((CD, DIM), lambda i: (0, 0)),
            pl.BlockSpec((CD, K), lambda i: (0, 0)),
        ],
        out_specs=pl.BlockSpec((_BLK,), lambda i: (i,)),
        out_shape=jax.ShapeDtypeStruct((N,), jnp.int32),
        scratch_shapes=[
            pltpu.VMEM((DIM, CD), jnp.float32),
            pltpu.VMEM((K, CD), jnp.float32),
            pltpu.VMEM((K, 1), jnp.float32),
        ],
        compiler_params=pltpu.CompilerParams(
            dimension_semantics=("arbitrary",)),
    )(xf, W, cbt)
    return out.reshape(B, L)


# speedup vs baseline: 1.0688x; 1.0688x over previous
import jax
import jax.numpy as jnp
from jax.experimental import pallas as pl
from jax.experimental.pallas import tpu as pltpu

_SB = 256
_G = 16


def _vq_kernel(x_hbm, w_ref, cbt_ref, out_ref,
               xbuf, wt_ref, cb_ref, c2_ref, sems):
    n_sb = x_hbm.shape[0] // _SB

    def copy_in(i, slot):
        return pltpu.make_async_copy(
            x_hbm.at[pl.ds(i * _SB, _SB), :], xbuf.at[slot], sems.at[slot])

    for j in range(_G):
        copy_in(j, j).start()

    wt_ref[...] = w_ref[...].T
    cb = cbt_ref[...].T
    cb_ref[...] = cb
    c2_ref[...] = jnp.sum(cb * cb, axis=1, keepdims=True)

    nbuf = 2 * _G
    for g in range(0, n_sb, _G):
        for j in range(_G):
            if g + _G + j < n_sb:
                copy_in(g + _G + j, (g + _G + j) % nbuf).start()
        for j in range(_G):
            copy_in(g + j, (g + j) % nbuf).wait()
        for j in range(_G):
            i = g + j
            h = jnp.dot(xbuf[i % nbuf], wt_ref[...],
                        preferred_element_type=jnp.float32)
            mean = jnp.mean(h, axis=-1, keepdims=True)
            hc = h - mean
            var = jnp.mean(hc * hc, axis=-1, keepdims=True)
            hn = hc * jax.lax.rsqrt(var + 1e-5)
            scores_t = jnp.dot(cb_ref[...], hn.T,
                               preferred_element_type=jnp.float32)
            val = c2_ref[...] - 2.0 * scores_t
            idx = jnp.argmin(val, axis=0)
            out_ref[pl.ds(i * _SB, _SB)] = idx.astype(jnp.int32)


@jax.jit
def kernel(x, W, codebook):
    B, L, DIM = x.shape
    K, CD = codebook.shape
    N = B * L
    xf = x.reshape(N, DIM)
    cbt = codebook.T
    out = pl.pallas_call(
        _vq_kernel,
        in_specs=[
            pl.BlockSpec(memory_space=pltpu.MemorySpace.HBM),
            pl.BlockSpec((CD, DIM), lambda: (0, 0)),
            pl.BlockSpec((CD, K), lambda: (0, 0)),
        ],
        out_specs=pl.BlockSpec((N,), lambda: (0,)),
        out_shape=jax.ShapeDtypeStruct((N,), jnp.int32),
        scratch_shapes=[
            pltpu.VMEM((2 * _G, _SB, DIM), jnp.float32),
            pltpu.VMEM((DIM, CD), jnp.float32),
            pltpu.VMEM((K, CD), jnp.float32),
            pltpu.VMEM((K, 1), jnp.float32),
            pltpu.SemaphoreType.DMA((2 * _G,)),
        ],
    )(xf, W, cbt)
    return out.reshape(B, L)


# speedup vs baseline: 1.1163x; 1.0445x over previous
import jax
import jax.numpy as jnp
from jax.experimental import pallas as pl
from jax.experimental.pallas import tpu as pltpu

_SB = 256
_G = 8


def _vq_kernel(x_hbm, w_ref, cbt_ref, out_ref,
               xbuf, wt_ref, cb_ref, c2_ref, sems):
    n_sb = x_hbm.shape[0] // _SB

    def copy_in(i, slot):
        return pltpu.make_async_copy(
            x_hbm.at[pl.ds(i * _SB, _SB), :], xbuf.at[slot], sems.at[slot])

    for j in range(_G):
        copy_in(j, j).start()

    wt_ref[...] = w_ref[...].T
    cb = cbt_ref[...].T
    cb_ref[...] = cb
    c2_ref[...] = jnp.sum(cb * cb, axis=1, keepdims=True)

    nbuf = 2 * _G
    for g in range(0, n_sb, _G):
        for j in range(_G):
            if g + _G + j < n_sb:
                copy_in(g + _G + j, (g + _G + j) % nbuf).start()
        for j in range(_G):
            copy_in(g + j, (g + j) % nbuf).wait()
        for j in range(_G):
            i = g + j
            h = jnp.dot(xbuf[i % nbuf], wt_ref[...],
                        preferred_element_type=jnp.float32)
            mean = jnp.mean(h, axis=-1, keepdims=True)
            hc = h - mean
            var = jnp.mean(hc * hc, axis=-1, keepdims=True)
            hn = hc * jax.lax.rsqrt(var + 1e-5)
            scores_t = jnp.dot(cb_ref[...], hn.T,
                               preferred_element_type=jnp.float32)
            val = c2_ref[...] - 2.0 * scores_t
            idx = jnp.argmin(val, axis=0)
            out_ref[pl.ds(i * _SB, _SB)] = idx.astype(jnp.int32)


@jax.jit
def kernel(x, W, codebook):
    B, L, DIM = x.shape
    K, CD = codebook.shape
    N = B * L
    xf = x.reshape(N, DIM)
    cbt = codebook.T
    out = pl.pallas_call(
        _vq_kernel,
        in_specs=[
            pl.BlockSpec(memory_space=pltpu.MemorySpace.HBM),
            pl.BlockSpec((CD, DIM), lambda: (0, 0)),
            pl.BlockSpec((CD, K), lambda: (0, 0)),
        ],
        out_specs=pl.BlockSpec((N,), lambda: (0,)),
        out_shape=jax.ShapeDtypeStruct((N,), jnp.int32),
        scratch_shapes=[
            pltpu.VMEM((2 * _G, _SB, DIM), jnp.float32),
            pltpu.VMEM((DIM, CD), jnp.float32),
            pltpu.VMEM((K, CD), jnp.float32),
            pltpu.VMEM((K, 1), jnp.float32),
            pltpu.SemaphoreType.DMA((2 * _G,)),
        ],
    )(xf, W, cbt)
    return out.reshape(B, L)


# speedup vs baseline: 1.1376x; 1.0191x over previous
import jax
import jax.numpy as jnp
from jax.experimental import pallas as pl
from jax.experimental.pallas import tpu as pltpu

_SB = 256
_GROUPS = [2, 2, 4, 8]


def _vq_kernel(x_hbm, w_ref, cbt_ref, out_ref,
               xbuf, wt_ref, cb_ref, c2_ref, sems):
    n_sb = x_hbm.shape[0] // _SB

    def copy_in(i, slot):
        return pltpu.make_async_copy(
            x_hbm.at[pl.ds(i * _SB, _SB), :], xbuf.at[slot], sems.at[slot])

    groups = []
    done = 0
    for g in _GROUPS:
        if done >= n_sb:
            break
        g = min(g, n_sb - done)
        groups.append((done, g))
        done += g
    while done < n_sb:
        g = min(_GROUPS[-1], n_sb - done)
        groups.append((done, g))
        done += g

    for start, size in groups[:2]:
        for i in range(start, start + size):
            copy_in(i, i).start()

    wt_ref[...] = w_ref[...].T
    cb = cbt_ref[...].T
    cb_ref[...] = cb
    c2_ref[...] = jnp.sum(cb * cb, axis=1, keepdims=True)

    for gi, (start, size) in enumerate(groups):
        if gi + 2 < len(groups):
            nstart, nsize = groups[gi + 2]
            for i in range(nstart, nstart + nsize):
                copy_in(i, i).start()
        for i in range(start, start + size):
            copy_in(i, i).wait()
        for i in range(start, start + size):
            h = jnp.dot(xbuf[i], wt_ref[...],
                        preferred_element_type=jnp.float32)
            mean = jnp.mean(h, axis=-1, keepdims=True)
            hc = h - mean
            var = jnp.mean(hc * hc, axis=-1, keepdims=True)
            hn = hc * jax.lax.rsqrt(var + 1e-5)
            scores_t = jnp.dot(cb_ref[...], hn.T,
                               preferred_element_type=jnp.float32)
            val = c2_ref[...] - 2.0 * scores_t
            idx = jnp.argmin(val, axis=0)
            out_ref[pl.ds(i * _SB, _SB)] = idx.astype(jnp.int32)


@jax.jit
def kernel(x, W, codebook):
    B, L, DIM = x.shape
    K, CD = codebook.shape
    N = B * L
    xf = x.reshape(N, DIM)
    cbt = codebook.T
    out = pl.pallas_call(
        _vq_kernel,
        in_specs=[
            pl.BlockSpec(memory_space=pltpu.MemorySpace.HBM),
            pl.BlockSpec((CD, DIM), lambda: (0, 0)),
            pl.BlockSpec((CD, K), lambda: (0, 0)),
        ],
        out_specs=pl.BlockSpec((N,), lambda: (0,)),
        out_shape=jax.ShapeDtypeStruct((N,), jnp.int32),
        scratch_shapes=[
            pltpu.VMEM((N // _SB, _SB, DIM), jnp.float32),
            pltpu.VMEM((DIM, CD), jnp.float32),
            pltpu.VMEM((K, CD), jnp.float32),
            pltpu.VMEM((K, 1), jnp.float32),
            pltpu.SemaphoreType.DMA((N // _SB,)),
        ],
    )(xf, W, cbt)
    return out.reshape(B, L)
